# baseline (speedup 1.0000x reference)
import functools

import jax
import jax.numpy as jnp
from jax import lax
from jax.experimental import pallas as pl
from jax.experimental.pallas import tpu as pltpu
from jax.experimental.pallas import tpu_sc as plsc

_LANES = 16
_CHUNK = 4096


def _sc_lookup(table_pad, z, charge):
    n = z.shape[0]
    info = plsc.get_sparse_core_info()
    nw = info.num_cores * info.num_subcores
    per_w = n // nw
    nchunks = per_w // _CHUNK
    tpad = table_pad.shape[0]
    mesh = plsc.VectorSubcoreMesh(core_axis_name="c", subcore_axis_name="s")

    @functools.partial(
        pl.kernel,
        mesh=mesh,
        out_type=jax.ShapeDtypeStruct((n,), jnp.float32),
        compiler_params=pltpu.CompilerParams(needs_layout_passes=False),
        scratch_types=[
            pltpu.VMEM((tpad,), jnp.float32),
            pltpu.VMEM((per_w,), jnp.int32),
            pltpu.VMEM((per_w,), jnp.int32),
            pltpu.VMEM((per_w,), jnp.float32),
            pltpu.SemaphoreType.DMA,
        ]
        + [pltpu.SemaphoreType.DMA] * nchunks,
    )
    def k(table_hbm, z_hbm, q_hbm, out_hbm, t_v, z_v, q_v, o_v, sem_o, *sems):
        wid = lax.axis_index("s") * info.num_cores + lax.axis_index("c")
        base = wid * per_w

        cp_z, cp_q = [], []
        for g in range(nchunks):
            lo = g * _CHUNK
            cp_z.append(
                pltpu.async_copy(
                    z_hbm.at[pl.ds(base + lo, _CHUNK)],
                    z_v.at[pl.ds(lo, _CHUNK)],
                    sems[g],
                )
            )
            cp_q.append(
                pltpu.async_copy(
                    q_hbm.at[pl.ds(base + lo, _CHUNK)],
                    q_v.at[pl.ds(lo, _CHUNK)],
                    sems[g],
                )
            )
        pltpu.sync_copy(table_hbm, t_v)

        cp_o = []
        for g in range(nchunks):
            lo = g * _CHUNK
            cp_z[g].wait()
            cp_q[g].wait()

            @plsc.parallel_loop(lo, lo + _CHUNK, _LANES, unroll=8)
            def body(off):
                z16 = z_v[pl.ds(off, _LANES)]
                q16 = q_v[pl.ds(off, _LANES)]
                idx = z16 * 3 + q16
                o_v[pl.ds(off, _LANES)] = plsc.load_gather(t_v, [idx])

            cp_o.append(
                pltpu.async_copy(
                    o_v.at[pl.ds(lo, _CHUNK)],
                    out_hbm.at[pl.ds(base + lo, _CHUNK)],
                    sem_o,
                )
            )
        for g in range(nchunks):
            cp_o[g].wait()

    return k(table_pad, z, charge)


def kernel(z, charge, energy_table):
    table_pad = jnp.zeros((128,), jnp.float32).at[:108].set(
        energy_table.reshape(-1)
    )
    return _sc_lookup(table_pad, z, charge)
